# baseline (speedup 1.0000x reference)
import functools

import jax
jax.config.update("jax_enable_x64", True)
import jax.numpy as jnp
from jax import lax
from jax.experimental import pallas as pl
from jax.experimental.pallas import tpu as pltpu
from jax.experimental.pallas import tpu_sc as plsc

N = 10000
WD = 32
NC, NS = 2, 16
NW = NC * NS
CH = 128
NB = 8
R = 10240
ROWS_PT = R // NS
EPS = 1e-3


def _sc_scatter_add(y, src3, dst3, zeros):
    n_chunks = src3.shape[1]
    mesh = plsc.VectorSubcoreMesh(
        core_axis_name="c", subcore_axis_name="s", num_cores=NC, num_subcores=NS
    )

    @functools.partial(
        pl.kernel,
        out_type=jax.ShapeDtypeStruct((NC, R, WD), jnp.float32),
        mesh=mesh,
        compiler_params=pltpu.CompilerParams(use_tc_tiling_on_sc=False),
        scratch_types=[
            pltpu.VMEM((n_chunks, CH), jnp.int32),
            pltpu.VMEM((n_chunks, CH), jnp.int32),
            pltpu.VMEM((NB, CH, WD), jnp.float32),
            pltpu.VMEM_SHARED((R, WD), jnp.float32),
            pltpu.SemaphoreType.DMA,
            pltpu.SemaphoreType.DMA,
        ],
    )
    def k(y_hbm, src_hbm, dst_hbm, zero_hbm, out_hbm, src_v, dst_v, gbuf, acc, gsem, ssem):
        c = lax.axis_index("c")
        s = lax.axis_index("s")
        wid = s * NC + c
        pltpu.sync_copy(src_hbm.at[wid], src_v)
        pltpu.sync_copy(dst_hbm.at[wid], dst_v)
        pltpu.sync_copy(zero_hbm, acc.at[pl.ds(s * ROWS_PT, ROWS_PT)])
        plsc.subcore_barrier()

        def group(jj, base):
            gds = [
                pltpu.async_copy(y_hbm.at[src_v.at[base + jnp.int32(b)]],
                                 gbuf.at[jnp.int32(b)], gsem)
                for b in range(NB)
            ]
            sds = []
            for b in range(NB):
                gds[b].wait()
                sds.append(
                    pltpu.async_copy(gbuf.at[jnp.int32(b)],
                                     acc.at[dst_v.at[base + jnp.int32(b)]],
                                     ssem, add=True)
                )
            for d in sds:
                d.wait()
            return base + jnp.int32(NB)

        lax.fori_loop(0, n_chunks // NB, group, jnp.int32(0))
        plsc.subcore_barrier()
        pltpu.sync_copy(
            acc.at[pl.ds(s * ROWS_PT, ROWS_PT)],
            out_hbm.at[c, pl.ds(s * ROWS_PT, ROWS_PT)],
        )

    return k(y, src3, dst3, zeros)


def _tc_pre(x, wn, ws):
    def body(x_ref, wn_ref, ws_ref, y_ref, s_ref):
        xx = x_ref[...]
        y_ref[...] = jnp.dot(xx, wn_ref[...], preferred_element_type=jnp.float32)
        s_ref[...] = jnp.dot(xx, ws_ref[...], preferred_element_type=jnp.float32)

    return pl.pallas_call(
        body,
        out_shape=(
            jax.ShapeDtypeStruct((N, WD), jnp.float32),
            jax.ShapeDtypeStruct((N, WD), jnp.float32),
        ),
    )(x, wn, ws)


def _bn_relu(h, g, b):
    mean = jnp.mean(h, axis=0, keepdims=True)
    var = jnp.mean((h - mean) ** 2, axis=0, keepdims=True)
    return jnp.maximum((h - mean) / jnp.sqrt(var + EPS) * g + b, 0.0)


def _tc_mid(s0, agg, g, b, wn, ws):
    def body(s_ref, a_ref, g_ref, b_ref, wn_ref, ws_ref, y_ref, so_ref):
        h = s_ref[...] + a_ref[0, :N, :] + a_ref[1, :N, :]
        h = _bn_relu(h, g_ref[...], b_ref[...])
        y_ref[...] = jnp.dot(h, wn_ref[...], preferred_element_type=jnp.float32)
        so_ref[...] = jnp.dot(h, ws_ref[...], preferred_element_type=jnp.float32)

    return pl.pallas_call(
        body,
        out_shape=(
            jax.ShapeDtypeStruct((N, WD), jnp.float32),
            jax.ShapeDtypeStruct((N, WD), jnp.float32),
        ),
    )(s0, agg, g, b, wn, ws)


def _tc_post(s1, agg, g, b, wsem, bsem):
    def body(s_ref, a_ref, g_ref, b_ref, wsem_ref, bsem_ref, o_ref):
        h = s_ref[...] + a_ref[0, :N, :] + a_ref[1, :N, :]
        h = _bn_relu(h, g_ref[...], b_ref[...])
        z = jnp.dot(h, wsem_ref[...], preferred_element_type=jnp.float32)
        o_ref[...] = jax.nn.sigmoid(z + bsem_ref[...])

    return pl.pallas_call(
        body,
        out_shape=jax.ShapeDtypeStruct((N, 1), jnp.float32),
    )(s1, agg, g, b, wsem, bsem)


def kernel(x, edge_index, W_self0, W_neigh0, gamma0, beta0,
           W_self1, W_neigh1, gamma1, beta1, W_sem, b_sem):
    x = x.astype(jnp.float32)
    src = edge_index[0].astype(jnp.int32)
    dst = edge_index[1].astype(jnp.int32)
    E = src.shape[0]
    per = NW * CH
    n_chunks = -(-E // per)
    n_chunks = -(-n_chunks // NB) * NB
    epad = n_chunks * per - E
    src3 = jnp.pad(src, (0, epad)).reshape(NW, n_chunks, CH)
    pad_dst = N + (jnp.arange(epad, dtype=jnp.int32) % (R - N))
    dst3 = jnp.concatenate([dst, pad_dst]).reshape(NW, n_chunks, CH)
    zeros = jnp.zeros((ROWS_PT, WD), jnp.float32)

    g0 = gamma0.reshape(1, WD).astype(jnp.float32)
    b0 = beta0.reshape(1, WD).astype(jnp.float32)
    g1 = gamma1.reshape(1, WD).astype(jnp.float32)
    b1 = beta1.reshape(1, WD).astype(jnp.float32)
    bs = b_sem.reshape(1, 1).astype(jnp.float32)

    y0, s0 = _tc_pre(x, W_neigh0.astype(jnp.float32), W_self0.astype(jnp.float32))
    agg0 = _sc_scatter_add(y0, src3, dst3, zeros)
    y1, s1 = _tc_mid(s0, agg0, g0, b0,
                     W_neigh1.astype(jnp.float32), W_self1.astype(jnp.float32))
    agg1 = _sc_scatter_add(y1, src3, dst3, zeros)
    out = _tc_post(s1, agg1, g1, b1, W_sem.astype(jnp.float32), bs)
    return out.astype(jnp.float64)


# speedup vs baseline: 1.8377x; 1.8377x over previous
import functools

import jax
jax.config.update("jax_enable_x64", True)
import jax.numpy as jnp
from jax import lax
from jax.experimental import pallas as pl
from jax.experimental.pallas import tpu as pltpu
from jax.experimental.pallas import tpu_sc as plsc

N = 10000
WD = 32
NC, NS = 2, 16
NW = NC * NS
CH = 128
NB = 8
R = 10240
ROWS_PT = R // NS
EPS = 1e-3


def _sc_scatter_add(y, src3, dst3, zeros):
    n_chunks = src3.shape[1]
    mesh = plsc.VectorSubcoreMesh(
        core_axis_name="c", subcore_axis_name="s", num_cores=NC, num_subcores=NS
    )

    @functools.partial(
        pl.kernel,
        out_type=jax.ShapeDtypeStruct((NC, R, WD), jnp.float32),
        mesh=mesh,
        compiler_params=pltpu.CompilerParams(use_tc_tiling_on_sc=False),
        scratch_types=[
            pltpu.VMEM((n_chunks, CH), jnp.int32),
            pltpu.VMEM((n_chunks, CH), jnp.int32),
            pltpu.VMEM((NB, CH, WD), jnp.float32),
            pltpu.VMEM_SHARED((R, WD), jnp.float32),
            pltpu.SemaphoreType.DMA,
            pltpu.SemaphoreType.DMA,
        ],
    )
    def k(y_hbm, src_hbm, dst_hbm, zero_hbm, out_hbm, src_v, dst_v, gbuf, acc, gsem, ssem):
        c = lax.axis_index("c")
        s = lax.axis_index("s")
        wid = s * NC + c
        pltpu.sync_copy(src_hbm.at[wid], src_v)
        pltpu.sync_copy(dst_hbm.at[wid], dst_v)
        pltpu.sync_copy(zero_hbm, acc.at[pl.ds(s * ROWS_PT, ROWS_PT)])
        plsc.subcore_barrier()

        def group(jj, base):
            gds = [
                pltpu.async_copy(y_hbm.at[src_v.at[base + jnp.int32(b)]],
                                 gbuf.at[jnp.int32(b)], gsem)
                for b in range(NB)
            ]
            sds = []
            for b in range(NB):
                gds[b].wait()
                sds.append(
                    pltpu.async_copy(gbuf.at[jnp.int32(b)],
                                     acc.at[dst_v.at[base + jnp.int32(b)]],
                                     ssem, add=True)
                )
            for d in sds:
                d.wait()
            return base + jnp.int32(NB)

        lax.fori_loop(0, n_chunks // NB, group, jnp.int32(0))
        plsc.subcore_barrier()
        pltpu.sync_copy(
            acc.at[pl.ds(s * ROWS_PT, ROWS_PT)],
            out_hbm.at[c, pl.ds(s * ROWS_PT, ROWS_PT)],
        )

    return k(y, src3, dst3, zeros)


def _tc_pre(x, wn, ws):
    def body(x_ref, wn_ref, ws_ref, y_ref, s_ref):
        xx = x_ref[...]
        y_ref[...] = jnp.dot(xx, wn_ref[...], preferred_element_type=jnp.float32)
        s_ref[...] = jnp.dot(xx, ws_ref[...], preferred_element_type=jnp.float32)

    return pl.pallas_call(
        body,
        out_shape=(
            jax.ShapeDtypeStruct((N, WD), jnp.float32),
            jax.ShapeDtypeStruct((N, WD), jnp.float32),
        ),
    )(x, wn, ws)


def _bn_relu(h, g, b):
    mean = jnp.mean(h, axis=0, keepdims=True)
    var = jnp.mean((h - mean) ** 2, axis=0, keepdims=True)
    return jnp.maximum((h - mean) / jnp.sqrt(var + EPS) * g + b, 0.0)


def _tc_mid(s0, agg, g, b, wn, ws):
    def body(s_ref, a_ref, g_ref, b_ref, wn_ref, ws_ref, y_ref, so_ref):
        h = s_ref[...] + a_ref[0, :N, :] + a_ref[1, :N, :]
        h = _bn_relu(h, g_ref[...], b_ref[...])
        y_ref[...] = jnp.dot(h, wn_ref[...], preferred_element_type=jnp.float32)
        so_ref[...] = jnp.dot(h, ws_ref[...], preferred_element_type=jnp.float32)

    return pl.pallas_call(
        body,
        out_shape=(
            jax.ShapeDtypeStruct((N, WD), jnp.float32),
            jax.ShapeDtypeStruct((N, WD), jnp.float32),
        ),
    )(s0, agg, g, b, wn, ws)


def _tc_post(s1, agg, g, b, wsem, bsem):
    def body(s_ref, a_ref, g_ref, b_ref, wsem_ref, bsem_ref, o_ref):
        h = s_ref[...] + a_ref[0, :N, :] + a_ref[1, :N, :]
        h = _bn_relu(h, g_ref[...], b_ref[...])
        z = jnp.dot(h, wsem_ref[...], preferred_element_type=jnp.float32)
        o_ref[...] = jax.nn.sigmoid(z + bsem_ref[...])

    return pl.pallas_call(
        body,
        out_shape=jax.ShapeDtypeStruct((N, 1), jnp.float32),
    )(s1, agg, g, b, wsem, bsem)


def kernel(x, edge_index, W_self0, W_neigh0, gamma0, beta0,
           W_self1, W_neigh1, gamma1, beta1, W_sem, b_sem):
    x = x.astype(jnp.float32)
    src = edge_index[0].astype(jnp.int32)
    dst = edge_index[1].astype(jnp.int32)
    E = src.shape[0]
    per = NW * CH
    n_chunks = -(-E // per)
    n_chunks = -(-n_chunks // NB) * NB
    epad = n_chunks * per - E
    pad_src = jnp.arange(epad, dtype=jnp.int32) * 37 % N
    pad_dst = N + (jnp.arange(epad, dtype=jnp.int32) % (R - N))
    src3 = jnp.concatenate([src, pad_src]).reshape(NW, n_chunks, CH)
    dst3 = jnp.concatenate([dst, pad_dst]).reshape(NW, n_chunks, CH)
    zeros = jnp.zeros((ROWS_PT, WD), jnp.float32)

    g0 = gamma0.reshape(1, WD).astype(jnp.float32)
    b0 = beta0.reshape(1, WD).astype(jnp.float32)
    g1 = gamma1.reshape(1, WD).astype(jnp.float32)
    b1 = beta1.reshape(1, WD).astype(jnp.float32)
    bs = b_sem.reshape(1, 1).astype(jnp.float32)

    y0, s0 = _tc_pre(x, W_neigh0.astype(jnp.float32), W_self0.astype(jnp.float32))
    agg0 = _sc_scatter_add(y0, src3, dst3, zeros)
    y1, s1 = _tc_mid(s0, agg0, g0, b0,
                     W_neigh1.astype(jnp.float32), W_self1.astype(jnp.float32))
    agg1 = _sc_scatter_add(y1, src3, dst3, zeros)
    out = _tc_post(s1, agg1, g1, b1, W_sem.astype(jnp.float32), bs)
    return out.astype(jnp.float64)
